# Initial kernel scaffold; baseline (speedup 1.0000x reference)
import jax
import jax.numpy as jnp
from jax.experimental import pallas as pl


def kernel(features, coords, weight, bias):
    raise NotImplementedError("write your pallas kernel here")


# speedup vs baseline: 2.6456x; 2.6456x over previous
import functools

import jax
import jax.numpy as jnp
from jax.experimental import pallas as pl

D = H = W = 64
PD = D + 2
CIN = 64
CH = 32
CCOMB = CIN + CH
COUT = 4 * CH


def _conv_body(pz0, pz1, pz2, w_ref, b_ref, out_ref):
    acc = jnp.broadcast_to(b_ref[0], (D * H, COUT)).astype(jnp.float32)
    for dzi in range(3):
        ref = (pz0, pz1, pz2)[dzi]
        for dyi in range(3):
            for dxi in range(3):
                ko = (dzi * 3 + dyi) * 3 + dxi
                a = ref[0, dyi:dyi + D, dxi:dxi + W, :].reshape(D * W, CCOMB)
                acc = acc + jnp.dot(a, w_ref[ko],
                                    preferred_element_type=jnp.float32)
    out_ref[0] = acc.reshape(D, W, COUT)


def _conv(grid4, weight, bias):
    in_specs = [
        pl.BlockSpec((1, PD, PD, CCOMB), lambda z: (z, 0, 0, 0)),
        pl.BlockSpec((1, PD, PD, CCOMB), lambda z: (z + 1, 0, 0, 0)),
        pl.BlockSpec((1, PD, PD, CCOMB), lambda z: (z + 2, 0, 0, 0)),
        pl.BlockSpec((27, CCOMB, COUT), lambda z: (0, 0, 0)),
        pl.BlockSpec((1, COUT), lambda z: (0, 0)),
    ]
    return pl.pallas_call(
        _conv_body,
        grid=(D,),
        in_specs=in_specs,
        out_specs=pl.BlockSpec((1, H, W, COUT), lambda z: (z, 0, 0, 0)),
        out_shape=jax.ShapeDtypeStruct((D, H, W, COUT), jnp.float32),
    )(grid4, grid4, grid4, weight, bias.reshape(1, COUT))


def kernel(features, coords, weight, bias):
    T, N = features.shape[0], features.shape[1]
    z, y, x = coords[:, 0], coords[:, 1], coords[:, 2]
    lin64 = (z * H + y) * W + x
    plin = ((z + 1) * PD + (y + 1)) * PD + (x + 1)

    h = jnp.zeros((N, CH), dtype=jnp.float32)
    c = jnp.zeros((N, CH), dtype=jnp.float32)
    outs = []
    for t in range(T):
        comb = jnp.concatenate([features[t], h], axis=1)
        grid = jnp.zeros((PD * PD * PD, CCOMB), jnp.float32).at[plin].set(comb)
        convout = _conv(grid.reshape(PD, PD, PD, CCOMB), weight, bias)
        rows = convout.reshape(D * H * W, COUT)[lin64]
        cc_i = rows[:, 0 * CH:1 * CH]
        cc_f = rows[:, 1 * CH:2 * CH]
        cc_o = rows[:, 2 * CH:3 * CH]
        cc_g = rows[:, 3 * CH:4 * CH]
        i = jax.nn.sigmoid(cc_i)
        f = jax.nn.sigmoid(cc_f)
        o = jax.nn.sigmoid(cc_o)
        g = jnp.tanh(cc_g)
        c = f * c + i * g
        h = o * jnp.tanh(c)
        outs.append(h)
    return (jnp.stack(outs), h, c)


# speedup vs baseline: 2.6540x; 1.0032x over previous
import functools

import jax
import jax.numpy as jnp
from jax.experimental import pallas as pl

D = H = W = 64
PD = D + 2
CIN = 64
CH = 32
CCOMB = CIN + CH
COUT = 4 * CH


def _conv_body(pz0, pz1, pz2, w_ref, b_ref, out_ref):
    acc = jnp.broadcast_to(b_ref[0], (D * H, COUT)).astype(jnp.float32)
    for dzi in range(3):
        ref = (pz0, pz1, pz2)[dzi]
        plane = ref[0].astype(jnp.bfloat16)
        for dyi in range(3):
            for dxi in range(3):
                ko = (dzi * 3 + dyi) * 3 + dxi
                a = plane[dyi:dyi + D, dxi:dxi + W, :].reshape(D * W, CCOMB)
                acc = acc + jnp.dot(a, w_ref[ko].astype(jnp.bfloat16),
                                    preferred_element_type=jnp.float32)
    out_ref[0] = acc.reshape(D, W, COUT)


def _conv(grid4, weight, bias):
    in_specs = [
        pl.BlockSpec((1, PD, PD, CCOMB), lambda z: (z, 0, 0, 0)),
        pl.BlockSpec((1, PD, PD, CCOMB), lambda z: (z + 1, 0, 0, 0)),
        pl.BlockSpec((1, PD, PD, CCOMB), lambda z: (z + 2, 0, 0, 0)),
        pl.BlockSpec((27, CCOMB, COUT), lambda z: (0, 0, 0)),
        pl.BlockSpec((1, COUT), lambda z: (0, 0)),
    ]
    return pl.pallas_call(
        _conv_body,
        grid=(D,),
        in_specs=in_specs,
        out_specs=pl.BlockSpec((1, H, W, COUT), lambda z: (z, 0, 0, 0)),
        out_shape=jax.ShapeDtypeStruct((D, H, W, COUT), jnp.float32),
    )(grid4, grid4, grid4, weight, bias.reshape(1, COUT))


def kernel(features, coords, weight, bias):
    T, N = features.shape[0], features.shape[1]
    z, y, x = coords[:, 0], coords[:, 1], coords[:, 2]
    lin64 = (z * H + y) * W + x
    plin = ((z + 1) * PD + (y + 1)) * PD + (x + 1)

    h = jnp.zeros((N, CH), dtype=jnp.float32)
    c = jnp.zeros((N, CH), dtype=jnp.float32)
    outs = []
    for t in range(T):
        comb = jnp.concatenate([features[t], h], axis=1)
        grid = jnp.zeros((PD * PD * PD, CCOMB), jnp.float32).at[plin].set(comb)
        convout = _conv(grid.reshape(PD, PD, PD, CCOMB), weight, bias)
        rows = convout.reshape(D * H * W, COUT)[lin64]
        cc_i = rows[:, 0 * CH:1 * CH]
        cc_f = rows[:, 1 * CH:2 * CH]
        cc_o = rows[:, 2 * CH:3 * CH]
        cc_g = rows[:, 3 * CH:4 * CH]
        i = jax.nn.sigmoid(cc_i)
        f = jax.nn.sigmoid(cc_f)
        o = jax.nn.sigmoid(cc_o)
        g = jnp.tanh(cc_g)
        c = f * c + i * g
        h = o * jnp.tanh(c)
        outs.append(h)
    return (jnp.stack(outs), h, c)
